# baseline (speedup 1.0000x reference)
import functools

import jax
import jax.numpy as jnp
from jax import lax
from jax.experimental import pallas as pl
from jax.experimental.pallas import tpu as pltpu
from jax.experimental.pallas import tpu_sc as plsc

R = 16
N = 4096
L = 16
NCHUNK = N // L


def _sort_row_body(pos_hbm, val_hbm, outv_hbm, outk_hbm,
                   keys_v, vals_v, hist_v, rank_v, outk_v, outv_v):
  c = lax.axis_index("c")
  s = lax.axis_index("s")
  wid = s * 2 + c

  @pl.when(wid < R)
  def _():
    row = wid
    pltpu.sync_copy(pos_hbm.at[row], keys_v)
    pltpu.sync_copy(val_hbm.at[row], vals_v)

    cal = plsc.scan_count(lax.iota(jnp.int32, L))[0]

    UZ = 16
    def zero_body(i, carry):
      for u in range(UZ):
        hist_v[pl.ds((i * UZ + u) * L, L)] = jnp.zeros((L,), jnp.int32)
      return carry

    lax.fori_loop(0, NCHUNK // UZ, zero_body, jnp.int32(0))

    U1 = 8
    def hist_body(i, carry):
      for u in range(U1):
        off = (i * U1 + u) * L
        d = keys_v[pl.ds(off, L)]
        cnt, lastm = plsc.scan_count(d)
        occ = cnt - cal
        prev = plsc.load_gather(hist_v, [d])
        rank_v[pl.ds(off, L)] = prev + occ
        plsc.addupdate_scatter(hist_v, [d], occ + 1, mask=lastm)
      return carry

    lax.fori_loop(0, NCHUNK // U1, hist_body, jnp.int32(0))

    U2 = 8
    def scan_body(i, carry):
      incl = [None] * U2
      hs = [None] * U2
      for u in range(U2):
        hs[u] = hist_v[pl.ds((i * U2 + u) * L, L)]
        incl[u] = plsc.cumsum(hs[u])
      for u in range(U2):
        hist_v[pl.ds((i * U2 + u) * L, L)] = incl[u] - hs[u] + carry
        carry = carry + jnp.max(incl[u])
      return carry

    lax.fori_loop(0, NCHUNK // U2, scan_body, jnp.int32(0))

    U3 = 8
    def perm_body(i, carry):
      for u in range(U3):
        off = (i * U3 + u) * L
        d = keys_v[pl.ds(off, L)]
        v = vals_v[pl.ds(off, L)]
        r = rank_v[pl.ds(off, L)]
        base = plsc.load_gather(hist_v, [d])
        pos = base + r
        plsc.store_scatter(outv_v, [pos], v)
        plsc.store_scatter(outk_v, [pos], d)
      return carry

    lax.fori_loop(0, NCHUNK // U3, perm_body, jnp.int32(0))

    pltpu.sync_copy(outv_v, outv_hbm.at[row])
    pltpu.sync_copy(outk_v, outk_hbm.at[row])


@jax.jit
def kernel(indices, position_indices):
  mesh = plsc.VectorSubcoreMesh(core_axis_name="c", subcore_axis_name="s")
  run = pl.kernel(
      _sort_row_body,
      out_type=(
          jax.ShapeDtypeStruct((R, N), jnp.float32),
          jax.ShapeDtypeStruct((R, N), jnp.int32),
      ),
      mesh=mesh,
      compiler_params=pltpu.CompilerParams(needs_layout_passes=False),
      scratch_types=[
          pltpu.VMEM((N,), jnp.int32),
          pltpu.VMEM((N,), jnp.float32),
          pltpu.VMEM((N,), jnp.int32),
          pltpu.VMEM((N,), jnp.int32),
          pltpu.VMEM((N,), jnp.int32),
          pltpu.VMEM((N,), jnp.float32),
      ],
  )
  sorted_vals, sorted_keys = run(position_indices, indices)
  return sorted_vals, sorted_keys


# speedup vs baseline: 1.0263x; 1.0263x over previous
import functools

import jax
import jax.numpy as jnp
from jax import lax
from jax.experimental import pallas as pl
from jax.experimental.pallas import tpu as pltpu
from jax.experimental.pallas import tpu_sc as plsc

R = 16
N = 4096
L = 16
NCHUNK = N // L


def _sort_row_body(pos_hbm, val_hbm, outv_hbm, outk_hbm,
                   keys_v, vals_v, hist_v, rank_v, outk_v, outv_v,
                   sem_k, sem_v, sem_o):
  c = lax.axis_index("c")
  s = lax.axis_index("s")
  wid = s * 2 + c

  @pl.when(wid < R)
  def _():
    row = wid
    h_k = pltpu.make_async_copy(pos_hbm.at[row], keys_v, sem_k)
    h_k.start()
    h_v = pltpu.make_async_copy(val_hbm.at[row], vals_v, sem_v)
    h_v.start()

    cal = plsc.scan_count(lax.iota(jnp.int32, L))[0]

    UZ = 16
    def zero_body(i, carry):
      for u in range(UZ):
        hist_v[pl.ds((i * UZ + u) * L, L)] = jnp.zeros((L,), jnp.int32)
      return carry

    lax.fori_loop(0, NCHUNK // UZ, zero_body, jnp.int32(0))
    h_k.wait()

    U1 = 8
    def hist_body(i, carry):
      for u in range(U1):
        off = (i * U1 + u) * L
        d = keys_v[pl.ds(off, L)]
        cnt, lastm = plsc.scan_count(d)
        occ = cnt - cal
        prev = plsc.load_gather(hist_v, [d])
        rank_v[pl.ds(off, L)] = prev + occ
        plsc.addupdate_scatter(hist_v, [d], occ + 1, mask=lastm)
      return carry

    lax.fori_loop(0, NCHUNK // U1, hist_body, jnp.int32(0))

    U2 = 8
    def scan_body(i, carry):
      incl = [None] * U2
      hs = [None] * U2
      for u in range(U2):
        hs[u] = hist_v[pl.ds((i * U2 + u) * L, L)]
        incl[u] = plsc.cumsum(hs[u])
      for u in range(U2):
        hist_v[pl.ds((i * U2 + u) * L, L)] = incl[u] - hs[u] + carry
        carry = carry + incl[u][L - 1]
      return carry

    lax.fori_loop(0, NCHUNK // U2, scan_body, jnp.int32(0))
    h_v.wait()

    U3 = 8
    def perm_body(i, carry):
      for u in range(U3):
        off = (i * U3 + u) * L
        d = keys_v[pl.ds(off, L)]
        v = vals_v[pl.ds(off, L)]
        r = rank_v[pl.ds(off, L)]
        base = plsc.load_gather(hist_v, [d])
        pos = base + r
        plsc.store_scatter(outv_v, [pos], v)
        plsc.store_scatter(outk_v, [pos], d)
      return carry

    lax.fori_loop(0, NCHUNK // U3, perm_body, jnp.int32(0))

    h_ov = pltpu.make_async_copy(outv_v, outv_hbm.at[row], sem_o)
    h_ov.start()
    pltpu.sync_copy(outk_v, outk_hbm.at[row])
    h_ov.wait()


@jax.jit
def kernel(indices, position_indices):
  mesh = plsc.VectorSubcoreMesh(core_axis_name="c", subcore_axis_name="s")
  run = pl.kernel(
      _sort_row_body,
      out_type=(
          jax.ShapeDtypeStruct((R, N), jnp.float32),
          jax.ShapeDtypeStruct((R, N), jnp.int32),
      ),
      mesh=mesh,
      compiler_params=pltpu.CompilerParams(needs_layout_passes=False),
      scratch_types=[
          pltpu.VMEM((N,), jnp.int32),
          pltpu.VMEM((N,), jnp.float32),
          pltpu.VMEM((N,), jnp.int32),
          pltpu.VMEM((N,), jnp.int32),
          pltpu.VMEM((N,), jnp.int32),
          pltpu.VMEM((N,), jnp.float32),
          pltpu.SemaphoreType.DMA,
          pltpu.SemaphoreType.DMA,
          pltpu.SemaphoreType.DMA,
      ],
  )
  sorted_vals, sorted_keys = run(position_indices, indices)
  return sorted_vals, sorted_keys


# speedup vs baseline: 1.0608x; 1.0336x over previous
import functools

import jax
import jax.numpy as jnp
from jax import lax
from jax.experimental import pallas as pl
from jax.experimental.pallas import tpu as pltpu
from jax.experimental.pallas import tpu_sc as plsc

R = 16
N = 4096
L = 16
NCHUNK = N // L


def _sort_row_body(pos_hbm, val_hbm, outv_hbm, outk_hbm,
                   keys_v, vals_v, hist_v, rank_v, occm_v, outk_v, outv_v,
                   sem_k, sem_v, sem_o):
  c = lax.axis_index("c")
  s = lax.axis_index("s")
  wid = s * 2 + c

  @pl.when(wid < R)
  def _():
    row = wid
    h_k = pltpu.make_async_copy(pos_hbm.at[row], keys_v, sem_k)
    h_k.start()
    h_v = pltpu.make_async_copy(val_hbm.at[row], vals_v, sem_v)
    h_v.start()

    cal = plsc.scan_count(lax.iota(jnp.int32, L))[0]

    UZ = 16
    def zero_body(i, carry):
      for u in range(UZ):
        hist_v[pl.ds((i * UZ + u) * L, L)] = jnp.zeros((L,), jnp.int32)
      return carry

    lax.fori_loop(0, NCHUNK // UZ, zero_body, jnp.int32(0))
    h_k.wait()

    U1 = 8
    def scan_only_body(i, carry):
      for u in range(U1):
        off = (i * U1 + u) * L
        d = keys_v[pl.ds(off, L)]
        cnt, lastm = plsc.scan_count(d)
        occ = cnt - cal
        rank_v[pl.ds(off, L)] = occ
        occm_v[pl.ds(off, L)] = jnp.where(lastm, occ + 1, 0)
      return carry

    lax.fori_loop(0, NCHUNK // U1, scan_only_body, jnp.int32(0))

    def hist_body(i, carry):
      for u in range(U1):
        off = (i * U1 + u) * L
        d = keys_v[pl.ds(off, L)]
        om = occm_v[pl.ds(off, L)]
        prev = plsc.load_gather(hist_v, [d])
        rank_v[pl.ds(off, L)] = rank_v[pl.ds(off, L)] + prev
        plsc.addupdate_scatter(hist_v, [d], om, mask=om > 0)
      return carry

    lax.fori_loop(0, NCHUNK // U1, hist_body, jnp.int32(0))

    U2 = 8
    def scan_body(i, carry):
      incl = [None] * U2
      hs = [None] * U2
      for u in range(U2):
        hs[u] = hist_v[pl.ds((i * U2 + u) * L, L)]
        incl[u] = plsc.cumsum(hs[u])
      for u in range(U2):
        hist_v[pl.ds((i * U2 + u) * L, L)] = incl[u] - hs[u] + carry
        carry = carry + incl[u][L - 1]
      return carry

    lax.fori_loop(0, NCHUNK // U2, scan_body, jnp.int32(0))
    h_v.wait()

    U3 = 8
    def perm_body(i, carry):
      for u in range(U3):
        off = (i * U3 + u) * L
        d = keys_v[pl.ds(off, L)]
        v = vals_v[pl.ds(off, L)]
        r = rank_v[pl.ds(off, L)]
        base = plsc.load_gather(hist_v, [d])
        pos = base + r
        plsc.store_scatter(outv_v, [pos], v)
        plsc.store_scatter(outk_v, [pos], d)
      return carry

    lax.fori_loop(0, NCHUNK // U3, perm_body, jnp.int32(0))

    h_ov = pltpu.make_async_copy(outv_v, outv_hbm.at[row], sem_o)
    h_ov.start()
    pltpu.sync_copy(outk_v, outk_hbm.at[row])
    h_ov.wait()


@jax.jit
def kernel(indices, position_indices):
  mesh = plsc.VectorSubcoreMesh(core_axis_name="c", subcore_axis_name="s")
  run = pl.kernel(
      _sort_row_body,
      out_type=(
          jax.ShapeDtypeStruct((R, N), jnp.float32),
          jax.ShapeDtypeStruct((R, N), jnp.int32),
      ),
      mesh=mesh,
      compiler_params=pltpu.CompilerParams(needs_layout_passes=False),
      scratch_types=[
          pltpu.VMEM((N,), jnp.int32),
          pltpu.VMEM((N,), jnp.float32),
          pltpu.VMEM((N,), jnp.int32),
          pltpu.VMEM((N,), jnp.int32),
          pltpu.VMEM((N,), jnp.int32),
          pltpu.VMEM((N,), jnp.int32),
          pltpu.VMEM((N,), jnp.float32),
          pltpu.SemaphoreType.DMA,
          pltpu.SemaphoreType.DMA,
          pltpu.SemaphoreType.DMA,
      ],
  )
  sorted_vals, sorted_keys = run(position_indices, indices)
  return sorted_vals, sorted_keys


# speedup vs baseline: 1.1704x; 1.1034x over previous
import functools

import jax
import jax.numpy as jnp
from jax import lax
from jax.experimental import pallas as pl
from jax.experimental.pallas import tpu as pltpu
from jax.experimental.pallas import tpu_sc as plsc

R = 16
N = 4096
L = 16
NCHUNK = N // L


def _sort_row_body(pos_hbm, val_hbm, outv_hbm, outk_hbm,
                   keys_v, vals_v, hist_v, rank_v, occm_v, outk_v, outv_v,
                   sem_k, sem_v, sem_o):
  c = lax.axis_index("c")
  s = lax.axis_index("s")
  wid = s * 2 + c

  @pl.when(wid < R)
  def _():
    row = wid
    h_k = pltpu.make_async_copy(pos_hbm.at[row], keys_v, sem_k)
    h_k.start()
    h_v = pltpu.make_async_copy(val_hbm.at[row], vals_v, sem_v)
    h_v.start()

    cal = plsc.scan_count(lax.iota(jnp.int32, L))[0]

    @plsc.parallel_loop(0, N, step=L, unroll=16)
    def zero_body(i):
      hist_v[pl.ds(i, L)] = jnp.zeros((L,), jnp.int32)

    h_k.wait()

    U1 = 8

    @plsc.parallel_loop(0, N, step=L, unroll=U1)
    def scan_only_body(i):
      d = keys_v[pl.ds(i, L)]
      cnt, lastm = plsc.scan_count(d)
      occ = cnt - cal
      rank_v[pl.ds(i, L)] = occ
      occm_v[pl.ds(i, L)] = jnp.where(lastm, occ + 1, 0)

    def hist_body(i, carry):
      for u in range(U1):
        off = (i * U1 + u) * L
        d = keys_v[pl.ds(off, L)]
        om = occm_v[pl.ds(off, L)]
        prev = plsc.load_gather(hist_v, [d])
        rank_v[pl.ds(off, L)] = rank_v[pl.ds(off, L)] + prev
        plsc.addupdate_scatter(hist_v, [d], om, mask=om > 0)
      return carry

    lax.fori_loop(0, NCHUNK // U1, hist_body, jnp.int32(0))

    @plsc.parallel_loop(0, N, step=L, unroll=8, carry=jnp.int32(0))
    def scan_body(i, carry):
      h = hist_v[pl.ds(i, L)]
      incl = plsc.cumsum(h)
      hist_v[pl.ds(i, L)] = incl - h + carry
      return carry + incl[L - 1]

    h_v.wait()

    @plsc.parallel_loop(0, N, step=L, unroll=8)
    def perm_body(i):
      d = keys_v[pl.ds(i, L)]
      v = vals_v[pl.ds(i, L)]
      r = rank_v[pl.ds(i, L)]
      base = plsc.load_gather(hist_v, [d])
      pos = base + r
      plsc.store_scatter(outv_v, [pos], v)
      plsc.store_scatter(outk_v, [pos], d)

    h_ov = pltpu.make_async_copy(outv_v, outv_hbm.at[row], sem_o)
    h_ov.start()
    pltpu.sync_copy(outk_v, outk_hbm.at[row])
    h_ov.wait()


@jax.jit
def kernel(indices, position_indices):
  mesh = plsc.VectorSubcoreMesh(core_axis_name="c", subcore_axis_name="s")
  run = pl.kernel(
      _sort_row_body,
      out_type=(
          jax.ShapeDtypeStruct((R, N), jnp.float32),
          jax.ShapeDtypeStruct((R, N), jnp.int32),
      ),
      mesh=mesh,
      compiler_params=pltpu.CompilerParams(needs_layout_passes=False),
      scratch_types=[
          pltpu.VMEM((N,), jnp.int32),
          pltpu.VMEM((N,), jnp.float32),
          pltpu.VMEM((N,), jnp.int32),
          pltpu.VMEM((N,), jnp.int32),
          pltpu.VMEM((N,), jnp.int32),
          pltpu.VMEM((N,), jnp.int32),
          pltpu.VMEM((N,), jnp.float32),
          pltpu.SemaphoreType.DMA,
          pltpu.SemaphoreType.DMA,
          pltpu.SemaphoreType.DMA,
      ],
  )
  sorted_vals, sorted_keys = run(position_indices, indices)
  return sorted_vals, sorted_keys
